# Initial kernel scaffold; baseline (speedup 1.0000x reference)
import jax
import jax.numpy as jnp
from jax.experimental import pallas as pl


def kernel(x, dx, ddx, enc_w0, enc_b0, enc_w1, enc_b1, enc_w2, enc_b2, dec_w0, dec_b0, dec_w1, dec_b1, dec_w2, dec_b2, E_w, E_b):
    raise NotImplementedError("write your pallas kernel here")


# speedup vs baseline: 15.9089x; 15.9089x over previous
import functools

import jax
import jax.numpy as jnp
from jax.experimental import pallas as pl
from jax.experimental.pallas import tpu as pltpu

N_ROWS = 65536
IN_DIM = 512
H1, H2 = 256, 128
LATENT = 3
SINDY_DIM = 22

P_CORES = 2
BLK_FWD = 512
BLK_STREAM = 1024

_F32 = jnp.float32


def _dot(a, b):
    return jnp.dot(a, b, preferred_element_type=_F32)


def _sindy_terms(zc):
    d = len(zc)
    ones = jnp.ones_like(zc[0])
    cols = [ones for _ in range(d)]
    cols += [zc[i] for i in range(d)]
    for i in range(d):
        for j in range(i, d):
            cols.append(zc[i] * zc[j])
    for i in range(d):
        for j in range(i, d):
            for k in range(j, d):
                cols.append(zc[i] * zc[j] * zc[k])
    return cols


def _fwd_kernel(x_ref, ew0, eb0, ew1, eb1, ew2, eb2,
                dw0, db0, dw1, db1, dw2, db2, Ew, Eb,
                z_ref, xb_ref, dzb_ref, ge_ref, gd_ref):
    j = pl.program_id(1)

    x = x_ref[...]
    h0 = jax.nn.sigmoid(_dot(x, ew0[...]) + eb0[...])
    g0 = h0 * (1.0 - h0)
    h1 = jax.nn.sigmoid(_dot(h0, ew1[...]) + eb1[...])
    g1 = h1 * (1.0 - h1)
    z = _dot(h1, ew2[...]) + eb2[...]
    z_ref[...] = z

    ge_blk = jax.lax.dot_general(g0, g1, (((0,), (0,)), ((), ())),
                                 preferred_element_type=_F32)

    @pl.when(j == 0)
    def _():
        ge_ref[...] = ge_blk[None]

    @pl.when(j != 0)
    def _():
        ge_ref[...] += ge_blk[None]

    zc = [z[:, i:i + 1] for i in range(LATENT)]
    terms = _sindy_terms(zc)
    acc = jnp.broadcast_to(Eb[...], z.shape)
    for t, term in enumerate(terms):
        acc = acc + term * Ew[t, :]
    dzb_ref[...] = acc

    hd0 = jax.nn.sigmoid(_dot(z, dw0[...]) + db0[...])
    gd0 = hd0 * (1.0 - hd0)
    hd1 = jax.nn.sigmoid(_dot(hd0, dw1[...]) + db1[...])
    gd1 = hd1 * (1.0 - hd1)
    xb_ref[...] = _dot(hd1, dw2[...]) + db2[...]

    gd_blk = jax.lax.dot_general(gd0, gd1, (((0,), (0,)), ((), ())),
                                 preferred_element_type=_F32)

    @pl.when(j == 0)
    def _():
        gd_ref[...] = gd_blk[None]

    @pl.when(j != 0)
    def _():
        gd_ref[...] += gd_blk[None]


def _jac_kernel(ge_ref, gd_ref, ew0, ew1, ew2, dw0, dw1, dw2,
                jet_ref, jdt_ref):
    inv_n = _F32(1.0 / N_ROWS)
    ge = (ge_ref[0] + ge_ref[1]) * inv_n
    jet_ref[...] = _dot(ew0[...], _dot(ew1[...] * ge, ew2[...]))
    gd = (gd_ref[0] + gd_ref[1]) * inv_n
    jdt_ref[...] = _dot(_dot(dw0[...], dw1[...] * gd), dw2[...])


def _stream_kernel(dx_ref, dzb_ref, jet_ref, jdt_ref, dz_ref, dxb_ref):
    dz_ref[...] = _dot(dx_ref[...], jet_ref[...])
    dxb_ref[...] = _dot(dzb_ref[...], jdt_ref[...])


def _full(shape):
    return pl.BlockSpec(shape, lambda *_: tuple(0 for _ in shape))


def kernel(x, dx, ddx, enc_w0, enc_b0, enc_w1, enc_b1, enc_w2, enc_b2,
           dec_w0, dec_b0, dec_w1, dec_b1, dec_w2, dec_b2, E_w, E_b,
           interpret=False):
    del ddx

    n = x.shape[0]
    jf = n // (P_CORES * BLK_FWD)
    row = lambda i, j: (i * jf + j, 0)

    z, xb, dzb, ge_parts, gd_parts = pl.pallas_call(
        _fwd_kernel,
        grid=(P_CORES, jf),
        in_specs=[
            pl.BlockSpec((BLK_FWD, IN_DIM), row),
            _full((IN_DIM, H1)), _full((H1,)),
            _full((H1, H2)), _full((H2,)),
            _full((H2, LATENT)), _full((LATENT,)),
            _full((LATENT, H2)), _full((H2,)),
            _full((H2, H1)), _full((H1,)),
            _full((H1, IN_DIM)), _full((IN_DIM,)),
            _full((SINDY_DIM, LATENT)), _full((LATENT,)),
        ],
        out_specs=[
            pl.BlockSpec((BLK_FWD, LATENT), row),
            pl.BlockSpec((BLK_FWD, IN_DIM), row),
            pl.BlockSpec((BLK_FWD, LATENT), row),
            pl.BlockSpec((1, H1, H2), lambda i, j: (i, 0, 0)),
            pl.BlockSpec((1, H2, H1), lambda i, j: (i, 0, 0)),
        ],
        out_shape=[
            jax.ShapeDtypeStruct((n, LATENT), _F32),
            jax.ShapeDtypeStruct((n, IN_DIM), _F32),
            jax.ShapeDtypeStruct((n, LATENT), _F32),
            jax.ShapeDtypeStruct((P_CORES, H1, H2), _F32),
            jax.ShapeDtypeStruct((P_CORES, H2, H1), _F32),
        ],
        compiler_params=pltpu.CompilerParams(
            dimension_semantics=("parallel", "arbitrary")),
        name="sindy_forward",
        interpret=interpret,
    )(x, enc_w0, enc_b0, enc_w1, enc_b1, enc_w2, enc_b2,
      dec_w0, dec_b0, dec_w1, dec_b1, dec_w2, dec_b2, E_w, E_b)

    jet, jdt = pl.pallas_call(
        _jac_kernel,
        out_shape=[
            jax.ShapeDtypeStruct((IN_DIM, LATENT), _F32),
            jax.ShapeDtypeStruct((LATENT, IN_DIM), _F32),
        ],
        name="sindy_mean_jac",
        interpret=interpret,
    )(ge_parts, gd_parts, enc_w0, enc_w1, enc_w2, dec_w0, dec_w1, dec_w2)

    js = n // (P_CORES * BLK_STREAM)
    srow = lambda i, j: (i * js + j, 0)
    dz, dxb = pl.pallas_call(
        _stream_kernel,
        grid=(P_CORES, js),
        in_specs=[
            pl.BlockSpec((BLK_STREAM, IN_DIM), srow),
            pl.BlockSpec((BLK_STREAM, LATENT), srow),
            _full((IN_DIM, LATENT)),
            _full((LATENT, IN_DIM)),
        ],
        out_specs=[
            pl.BlockSpec((BLK_STREAM, LATENT), srow),
            pl.BlockSpec((BLK_STREAM, IN_DIM), srow),
        ],
        out_shape=[
            jax.ShapeDtypeStruct((n, LATENT), _F32),
            jax.ShapeDtypeStruct((n, IN_DIM), _F32),
        ],
        compiler_params=pltpu.CompilerParams(
            dimension_semantics=("parallel", "arbitrary")),
        name="sindy_stream",
        interpret=interpret,
    )(dx, dzb, jet, jdt)

    return (z, dz, dzb, xb, dxb)
